# baseline (speedup 1.0000x reference)
import jax
import jax.numpy as jnp
from jax import lax
from jax.experimental import pallas as pl
from jax.experimental.pallas import tpu as pltpu
from jax.experimental.pallas import tpu_sc as plsc

B, C, H, W = 16, 20, 128, 128
N, DIN, D = 512, 256, 32
HW = H * W
NQ = B // 4
HWT = 8192
QROWS_PAD = HW + 64
SC_ROWS = NQ * QROWS_PAD * 4
QBASE = QROWS_PAD * 4
NWORK = 32
ZROWS = SC_ROWS // NWORK
ZBUF = 1024


def _ka_body(emb_ref, fidx_ref, w_ref, b_ref, proj_ref, ridx_ref):
    b = pl.program_id(0)
    q = b // 4
    j = b % 4
    emb = emb_ref[0]
    proj = lax.dot_general(
        emb, w_ref[...], (((1,), (0,)), ((), ())),
        preferred_element_type=jnp.float32,
        precision=lax.Precision.HIGHEST,
    ) + b_ref[...]
    proj_ref[0] = jnp.maximum(proj, 0.0)

    row = fidx_ref[0]
    ridx_ref[0] = (q * QBASE + 4 * row + j).reshape(4, 128)


def _ksc_body(proj_hbm, didx_hbm, scat_hbm,
              zbuf, didx_v, idxb, pbuf, wmap, val_v, sem):
    c = lax.axis_index("c")
    s = lax.axis_index("s")
    w = c * 16 + s

    z = jnp.zeros((16,), jnp.float32)

    def zstep(i, carry):
        for r in range(8):
            for g in range(2):
                zbuf[i * 8 + r, pl.ds(g * 16, 16)] = z
        return carry

    lax.fori_loop(0, ZBUF // 8, zstep, 0)
    zbase = w * ZROWS
    cps = [pltpu.async_copy(zbuf, scat_hbm.at[pl.ds(zbase + k * ZBUF, ZBUF)], sem)
           for k in range(8)]
    rem = ZROWS - 8 * ZBUF
    cps.append(pltpu.async_copy(zbuf.at[pl.ds(0, rem)],
                                scat_hbm.at[pl.ds(zbase + 8 * ZBUF, rem)], sem))

    b = c * 8 + s
    qb = (b // 4) * QBASE
    dump = qb + 4 * HW + b % 4

    @pl.when(s < 8)
    def _dedup():
        pltpu.sync_copy(didx_hbm.at[b], didx_v)
        pltpu.sync_copy(proj_hbm.at[b], val_v)
        iota = lax.iota(jnp.int32, 16)
        for g in range(N // 16):
            dv = didx_v[g // 8, pl.ds((g % 8) * 16, 16)]
            pbuf[pl.ds(g * 16, 16)] = lax.shift_right_logical(dv - qb, 2)
        for g in range(N // 16):
            pv = pbuf[pl.ds(g * 16, 16)]
            nv = iota + g * 16
            for i in range(16):
                plsc.store_scatter(wmap, [pv], nv, mask=iota == i)
        for g in range(N // 16):
            p16 = pbuf[pl.ds(g * 16, 16)]
            wn = plsc.load_gather(wmap, [p16])
            keep = wn == iota + g * 16
            dv = didx_v[g // 8, pl.ds((g % 8) * 16, 16)]
            idxb[g // 8, pl.ds((g % 8) * 16, 16)] = jnp.where(keep, dv, dump)

    for cp in cps:
        cp.wait()

    plsc.subcore_barrier()

    @pl.when(s < 8)
    def _scatter():
        cps2 = [pltpu.async_copy(val_v.at[pl.ds(k * 128, 128)],
                                 scat_hbm.at[idxb.at[k]], sem)
                for k in range(4)]
        for cp in cps2:
            cp.wait()


def _kc_body(scat_ref, sp_ref, out_ref):
    t = jnp.swapaxes(scat_ref[0], 0, 1)
    res = t.reshape(128, HWT // W, W)
    for j in range(4):
        out_ref[j, :C] = sp_ref[j]
        out_ref[j, C:] = res[j * 32:(j + 1) * 32]


def kernel(spatial_info, entity_embeddings, locations, W_proj, b_proj):
    lh = jnp.clip(locations[..., 0], 0, H - 1)
    lw = jnp.clip(locations[..., 1], 0, W - 1)
    fidx = (lh * W + lw).astype(jnp.int32).reshape(B, 1, N)
    b2 = b_proj.reshape(1, D)

    proj, ridx = pl.pallas_call(
        _ka_body,
        grid=(B,),
        in_specs=[
            pl.BlockSpec((1, N, DIN), lambda b: (b, 0, 0)),
            pl.BlockSpec((1, 1, N), lambda b: (b, 0, 0)),
            pl.BlockSpec((DIN, D), lambda b: (0, 0)),
            pl.BlockSpec((1, D), lambda b: (0, 0)),
        ],
        out_specs=[
            pl.BlockSpec((1, N, D), lambda b: (b, 0, 0)),
            pl.BlockSpec((1, 4, 128), lambda b: (b, 0, 0)),
        ],
        out_shape=[
            jax.ShapeDtypeStruct((B, N, D), jnp.float32),
            jax.ShapeDtypeStruct((B, 4, 128), jnp.int32),
        ],
    )(entity_embeddings, fidx, W_proj, b2)

    mesh = plsc.VectorSubcoreMesh(core_axis_name="c", subcore_axis_name="s")
    scat = pl.kernel(
        _ksc_body,
        out_type=jax.ShapeDtypeStruct((SC_ROWS, 32), jnp.float32),
        mesh=mesh,
        scratch_types=[
            pltpu.VMEM((ZBUF, 32), jnp.float32),
            pltpu.VMEM((4, 128), jnp.int32),
            pltpu.VMEM((4, 128), jnp.int32),
            pltpu.VMEM((N,), jnp.int32),
            pltpu.VMEM((HW,), jnp.int32),
            pltpu.VMEM((N, D), jnp.float32),
            pltpu.SemaphoreType.DMA,
        ],
        compiler_params=pltpu.CompilerParams(use_tc_tiling_on_sc=False,
                                             needs_layout_passes=False),
    )(proj, ridx)
    scatq = scat.reshape(NQ, QROWS_PAD, 128)

    ht = HWT // W
    out = pl.pallas_call(
        _kc_body,
        grid=(NQ, HW // HWT),
        in_specs=[
            pl.BlockSpec((1, HWT, 128), lambda q, h: (q, h, 0)),
            pl.BlockSpec((4, C, ht, W), lambda q, h: (q, 0, h, 0)),
        ],
        out_specs=pl.BlockSpec((4, C + D, ht, W), lambda q, h: (q, 0, h, 0)),
        out_shape=jax.ShapeDtypeStruct((B, C + D, H, W), jnp.float32),
    )(scatq, spatial_info)
    return out


# speedup vs baseline: 1.0670x; 1.0670x over previous
import jax
import jax.numpy as jnp
from jax import lax
from jax.experimental import pallas as pl
from jax.experimental.pallas import tpu as pltpu
from jax.experimental.pallas import tpu_sc as plsc

B, C, H, W = 16, 20, 128, 128
N, DIN, D = 512, 256, 32
HW = H * W
NQ = B // 4
HWT = 8192
QROWS_PAD = HW + 64
SC_ROWS = NQ * QROWS_PAD * 4
QBASE = QROWS_PAD * 4
NWORK = 32
ZROWS = SC_ROWS // NWORK
ZBUF = 1024


def _ka_body(emb_ref, fidx_ref, w_ref, b_ref, proj_ref, ridx_ref):
    q = pl.program_id(0)
    emb = emb_ref[...].reshape(4 * N, DIN)
    proj = lax.dot_general(
        emb, w_ref[...], (((1,), (0,)), ((), ())),
        preferred_element_type=jnp.float32,
        precision=lax.Precision.HIGHEST,
    ) + b_ref[...]
    proj_ref[...] = jnp.maximum(proj, 0.0).reshape(4, N, D)

    rows = fidx_ref[...]
    j = lax.broadcasted_iota(jnp.int32, (4, 1, N), 0)
    ridx_ref[...] = (q * QBASE + 4 * rows + j).reshape(4, 4, 128)


def _ksc_body(proj_hbm, didx_hbm, scat_hbm,
              zbuf, didx_v, idxb, pbuf, wmap, val_v, sem):
    c = lax.axis_index("c")
    s = lax.axis_index("s")
    w = c * 16 + s

    z = jnp.zeros((16,), jnp.float32)

    def zstep(i, carry):
        for r in range(8):
            for g in range(2):
                zbuf[i * 8 + r, pl.ds(g * 16, 16)] = z
        return carry

    lax.fori_loop(0, ZBUF // 8, zstep, 0)
    zbase = w * ZROWS
    cps = [pltpu.async_copy(zbuf, scat_hbm.at[pl.ds(zbase + k * ZBUF, ZBUF)], sem)
           for k in range(8)]
    rem = ZROWS - 8 * ZBUF
    cps.append(pltpu.async_copy(zbuf.at[pl.ds(0, rem)],
                                scat_hbm.at[pl.ds(zbase + 8 * ZBUF, rem)], sem))

    b = c * 8 + s
    qb = (b // 4) * QBASE
    dump = qb + 4 * HW + b % 4

    @pl.when(s < 8)
    def _dedup():
        pltpu.sync_copy(didx_hbm.at[b], didx_v)
        pltpu.sync_copy(proj_hbm.at[b], val_v)
        iota = lax.iota(jnp.int32, 16)
        for g in range(N // 16):
            dv = didx_v[g // 8, pl.ds((g % 8) * 16, 16)]
            pbuf[pl.ds(g * 16, 16)] = lax.shift_right_logical(dv - qb, 2)
        for g in range(N // 16):
            pv = pbuf[pl.ds(g * 16, 16)]
            nv = iota + g * 16
            for i in range(16):
                plsc.store_scatter(wmap, [pv], nv, mask=iota == i)
        for g in range(N // 16):
            p16 = pbuf[pl.ds(g * 16, 16)]
            wn = plsc.load_gather(wmap, [p16])
            keep = wn == iota + g * 16
            dv = didx_v[g // 8, pl.ds((g % 8) * 16, 16)]
            idxb[g // 8, pl.ds((g % 8) * 16, 16)] = jnp.where(keep, dv, dump)

    for cp in cps:
        cp.wait()

    plsc.subcore_barrier()

    @pl.when(s < 8)
    def _scatter():
        cps2 = [pltpu.async_copy(val_v.at[pl.ds(k * 128, 128)],
                                 scat_hbm.at[idxb.at[k]], sem)
                for k in range(4)]
        for cp in cps2:
            cp.wait()


def _kc_body(scat_ref, sp_ref, out_ref):
    t = jnp.swapaxes(scat_ref[0], 0, 1)
    res = t.reshape(128, HWT // W, W)
    for j in range(4):
        out_ref[j, :C] = sp_ref[j]
        out_ref[j, C:] = res[j * 32:(j + 1) * 32]


def kernel(spatial_info, entity_embeddings, locations, W_proj, b_proj):
    lh = jnp.clip(locations[..., 0], 0, H - 1)
    lw = jnp.clip(locations[..., 1], 0, W - 1)
    fidx = (lh * W + lw).astype(jnp.int32).reshape(B, 1, N)
    b2 = b_proj.reshape(1, D)

    proj, ridx = pl.pallas_call(
        _ka_body,
        grid=(NQ,),
        in_specs=[
            pl.BlockSpec((4, N, DIN), lambda q: (q, 0, 0)),
            pl.BlockSpec((4, 1, N), lambda q: (q, 0, 0)),
            pl.BlockSpec((DIN, D), lambda q: (0, 0)),
            pl.BlockSpec((1, D), lambda q: (0, 0)),
        ],
        out_specs=[
            pl.BlockSpec((4, N, D), lambda q: (q, 0, 0)),
            pl.BlockSpec((4, 4, 128), lambda q: (q, 0, 0)),
        ],
        out_shape=[
            jax.ShapeDtypeStruct((B, N, D), jnp.float32),
            jax.ShapeDtypeStruct((B, 4, 128), jnp.int32),
        ],
    )(entity_embeddings, fidx, W_proj, b2)

    mesh = plsc.VectorSubcoreMesh(core_axis_name="c", subcore_axis_name="s")
    scat = pl.kernel(
        _ksc_body,
        out_type=jax.ShapeDtypeStruct((SC_ROWS, 32), jnp.float32),
        mesh=mesh,
        scratch_types=[
            pltpu.VMEM((ZBUF, 32), jnp.float32),
            pltpu.VMEM((4, 128), jnp.int32),
            pltpu.VMEM((4, 128), jnp.int32),
            pltpu.VMEM((N,), jnp.int32),
            pltpu.VMEM((HW,), jnp.int32),
            pltpu.VMEM((N, D), jnp.float32),
            pltpu.SemaphoreType.DMA,
        ],
        compiler_params=pltpu.CompilerParams(use_tc_tiling_on_sc=False,
                                             needs_layout_passes=False),
    )(proj, ridx)
    scatq = scat.reshape(NQ, QROWS_PAD, 128)

    ht = HWT // W
    out = pl.pallas_call(
        _kc_body,
        grid=(NQ, HW // HWT),
        in_specs=[
            pl.BlockSpec((1, HWT, 128), lambda q, h: (q, h, 0)),
            pl.BlockSpec((4, C, ht, W), lambda q, h: (q, 0, h, 0)),
        ],
        out_specs=pl.BlockSpec((4, C + D, ht, W), lambda q, h: (q, 0, h, 0)),
        out_shape=jax.ShapeDtypeStruct((B, C + D, H, W), jnp.float32),
    )(scatq, spatial_info)
    return out


# speedup vs baseline: 1.0830x; 1.0150x over previous
import jax
import jax.numpy as jnp
from jax import lax
from jax.experimental import pallas as pl
from jax.experimental.pallas import tpu as pltpu
from jax.experimental.pallas import tpu_sc as plsc

B, C, H, W = 16, 20, 128, 128
N, DIN, D = 512, 256, 32
HW = H * W
NQ = B // 4
HWT = 16384
QROWS_PAD = HW + 64
SC_ROWS = NQ * QROWS_PAD * 4
QBASE = QROWS_PAD * 4
NWORK = 32
ZROWS = SC_ROWS // NWORK
ZBUF = 1024


def _ka_body(emb_ref, fidx_ref, w_ref, b_ref, proj_ref, ridx_ref):
    q = pl.program_id(0)
    emb = emb_ref[...].reshape(4 * N, DIN)
    proj = lax.dot_general(
        emb, w_ref[...], (((1,), (0,)), ((), ())),
        preferred_element_type=jnp.float32,
        precision=lax.Precision.HIGHEST,
    ) + b_ref[...]
    proj_ref[...] = jnp.maximum(proj, 0.0).reshape(4, N, D)

    rows = fidx_ref[...]
    j = lax.broadcasted_iota(jnp.int32, (4, 1, N), 0)
    ridx_ref[...] = (q * QBASE + 4 * rows + j).reshape(4, 4, 128)


def _ksc_body(proj_hbm, didx_hbm, scat_hbm,
              zbuf, didx_v, idxb, pbuf, wmap, val_v, sem):
    c = lax.axis_index("c")
    s = lax.axis_index("s")
    w = c * 16 + s

    z = jnp.zeros((16,), jnp.float32)

    def zstep(i, carry):
        for r in range(8):
            for g in range(2):
                zbuf[i * 8 + r, pl.ds(g * 16, 16)] = z
        return carry

    lax.fori_loop(0, ZBUF // 8, zstep, 0)
    zbase = w * ZROWS
    cps = [pltpu.async_copy(zbuf, scat_hbm.at[pl.ds(zbase + k * ZBUF, ZBUF)], sem)
           for k in range(8)]
    rem = ZROWS - 8 * ZBUF
    cps.append(pltpu.async_copy(zbuf.at[pl.ds(0, rem)],
                                scat_hbm.at[pl.ds(zbase + 8 * ZBUF, rem)], sem))

    b = c * 8 + s
    qb = (b // 4) * QBASE
    dump = qb + 4 * HW + b % 4

    @pl.when(s < 8)
    def _dedup():
        pltpu.sync_copy(didx_hbm.at[b], didx_v)
        pltpu.sync_copy(proj_hbm.at[b], val_v)
        iota = lax.iota(jnp.int32, 16)
        for g in range(N // 16):
            dv = didx_v[g // 8, pl.ds((g % 8) * 16, 16)]
            pbuf[pl.ds(g * 16, 16)] = lax.shift_right_logical(dv - qb, 2)
        for g in range(N // 16):
            pv = pbuf[pl.ds(g * 16, 16)]
            nv = iota + g * 16
            for i in range(16):
                plsc.store_scatter(wmap, [pv], nv, mask=iota == i)
        for g in range(N // 16):
            p16 = pbuf[pl.ds(g * 16, 16)]
            wn = plsc.load_gather(wmap, [p16])
            keep = wn == iota + g * 16
            dv = didx_v[g // 8, pl.ds((g % 8) * 16, 16)]
            idxb[g // 8, pl.ds((g % 8) * 16, 16)] = jnp.where(keep, dv, dump)

    for cp in cps:
        cp.wait()

    plsc.subcore_barrier()

    @pl.when(s < 8)
    def _scatter():
        cps2 = [pltpu.async_copy(val_v.at[pl.ds(k * 128, 128)],
                                 scat_hbm.at[idxb.at[k]], sem)
                for k in range(4)]
        for cp in cps2:
            cp.wait()


def _kc_body(scat_ref, sp_ref, out_ref):
    t = jnp.swapaxes(scat_ref[0], 0, 1)
    res = t.reshape(128, HWT // W, W)
    for j in range(4):
        out_ref[j, :C] = sp_ref[j]
        out_ref[j, C:] = res[j * 32:(j + 1) * 32]


def kernel(spatial_info, entity_embeddings, locations, W_proj, b_proj):
    lh = jnp.clip(locations[..., 0], 0, H - 1)
    lw = jnp.clip(locations[..., 1], 0, W - 1)
    fidx = (lh * W + lw).astype(jnp.int32).reshape(B, 1, N)
    b2 = b_proj.reshape(1, D)

    proj, ridx = pl.pallas_call(
        _ka_body,
        grid=(NQ,),
        in_specs=[
            pl.BlockSpec((4, N, DIN), lambda q: (q, 0, 0)),
            pl.BlockSpec((4, 1, N), lambda q: (q, 0, 0)),
            pl.BlockSpec((DIN, D), lambda q: (0, 0)),
            pl.BlockSpec((1, D), lambda q: (0, 0)),
        ],
        out_specs=[
            pl.BlockSpec((4, N, D), lambda q: (q, 0, 0)),
            pl.BlockSpec((4, 4, 128), lambda q: (q, 0, 0)),
        ],
        out_shape=[
            jax.ShapeDtypeStruct((B, N, D), jnp.float32),
            jax.ShapeDtypeStruct((B, 4, 128), jnp.int32),
        ],
    )(entity_embeddings, fidx, W_proj, b2)

    mesh = plsc.VectorSubcoreMesh(core_axis_name="c", subcore_axis_name="s")
    scat = pl.kernel(
        _ksc_body,
        out_type=jax.ShapeDtypeStruct((SC_ROWS, 32), jnp.float32),
        mesh=mesh,
        scratch_types=[
            pltpu.VMEM((ZBUF, 32), jnp.float32),
            pltpu.VMEM((4, 128), jnp.int32),
            pltpu.VMEM((4, 128), jnp.int32),
            pltpu.VMEM((N,), jnp.int32),
            pltpu.VMEM((HW,), jnp.int32),
            pltpu.VMEM((N, D), jnp.float32),
            pltpu.SemaphoreType.DMA,
        ],
        compiler_params=pltpu.CompilerParams(use_tc_tiling_on_sc=False,
                                             needs_layout_passes=False),
    )(proj, ridx)
    scatq = scat.reshape(NQ, QROWS_PAD, 128)

    ht = HWT // W
    out = pl.pallas_call(
        _kc_body,
        grid=(NQ, HW // HWT),
        in_specs=[
            pl.BlockSpec((1, HWT, 128), lambda q, h: (q, h, 0)),
            pl.BlockSpec((4, C, ht, W), lambda q, h: (q, 0, h, 0)),
        ],
        out_specs=pl.BlockSpec((4, C + D, ht, W), lambda q, h: (q, 0, h, 0)),
        out_shape=jax.ShapeDtypeStruct((B, C + D, H, W), jnp.float32),
    )(scatq, spatial_info)
    return out
